# baseline (speedup 1.0000x reference)
import functools

import jax
import jax.numpy as jnp
from jax import lax
from jax.experimental import pallas as pl
from jax.experimental.pallas import tpu as pltpu
from jax.experimental.pallas import tpu_sc as plsc

N_SRC = 10000
N_NEIGH = 160000
D_FEAT = 256
AGG = 128
K = N_NEIGH // N_SRC

N_TC = 9200
N_SC = N_SRC - N_TC

NC = 2
NS = 16
NW = NC * NS
L = 16

CH = 8
N_CHUNKS = N_SC // CH
CHUNKS_PER_W = -(-N_CHUNKS // NW)

_sc_mesh = plsc.VectorSubcoreMesh(core_axis_name="c", subcore_axis_name="s")


@functools.partial(
    pl.kernel,
    mesh=_sc_mesh,
    out_type=jax.ShapeDtypeStruct((N_SC, D_FEAT), jnp.float32),
    scratch_types=[
        pltpu.VMEM((2, CH * K, D_FEAT), jnp.float32),
        pltpu.VMEM((2, CH, D_FEAT), jnp.float32),
        pltpu.SemaphoreType.DMA,
        pltpu.SemaphoreType.DMA,
        pltpu.SemaphoreType.DMA,
        pltpu.SemaphoreType.DMA,
    ],
)
def _sc_mean(neigh_hbm, out_hbm, buf, obuf, si0, si1, so0, so1):
    wid = lax.axis_index("s") * NC + lax.axis_index("c")
    base = wid * CHUNKS_PER_W
    sin = (si0, si1)
    sout = (so0, so1)

    def src_slab(k):
        g = jnp.minimum(base + k, N_CHUNKS - 1)
        return neigh_hbm.at[pl.ds(N_TC * K + g * (CH * K), CH * K)]

    def dst_slab(k):
        g = jnp.minimum(base + k, N_CHUNKS - 1)
        return out_hbm.at[pl.ds(g * CH, CH)]

    def compute_chunk(p):
        bp = buf.at[p]
        op = obuf.at[p]

        def row_body(i, cc):
            r = i * K
            for c in range(D_FEAT // L):
                off = c * L
                acc = bp[r, pl.ds(off, L)]
                for j in range(1, K):
                    acc = acc + bp[r + j, pl.ds(off, L)]
                op[i, pl.ds(off, L)] = acc * (1.0 / K)
            return cc

        lax.fori_loop(0, CH, row_body, 0)

    for p in range(2):
        pltpu.async_copy(src_slab(p), buf.at[p], sin[p])

    def pair_body(t, carry):
        for p in range(2):
            k = 2 * t + p
            pltpu.make_async_copy(src_slab(k), buf.at[p], sin[p]).wait()
            @pl.when(t > 0)
            def _():
                pltpu.make_async_copy(obuf.at[p], dst_slab(k), sout[p]).wait()

            compute_chunk(p)
            pltpu.async_copy(obuf.at[p], dst_slab(k), sout[p])
            pltpu.async_copy(src_slab(k + 2), buf.at[p], sin[p])
        return carry

    PAIRS = CHUNKS_PER_W // 2
    lax.fori_loop(0, PAIRS, pair_body, 0)

    if CHUNKS_PER_W % 2:
        k = 2 * PAIRS
        pltpu.make_async_copy(src_slab(k), buf.at[0], sin[0]).wait()
        if PAIRS > 0:
            pltpu.make_async_copy(obuf.at[0], dst_slab(k), sout[0]).wait()
        compute_chunk(0)
        pltpu.async_copy(obuf.at[0], dst_slab(k), sout[0])
        pltpu.make_async_copy(src_slab(0), buf.at[1], sin[1]).wait()
        pltpu.make_async_copy(obuf.at[0], dst_slab(k), sout[0]).wait()
        if PAIRS > 0:
            pltpu.make_async_copy(obuf.at[1], dst_slab(0), sout[1]).wait()
    else:
        for p in range(2):
            pltpu.make_async_copy(src_slab(p), buf.at[p], sin[p]).wait()
            pltpu.make_async_copy(obuf.at[p], dst_slab(p), sout[p]).wait()


BLK_A = 920
BLK_B = 400


def _fused_body(neigh_ref, src_ref, ws_ref, wn_ref, out_ref):
    x = neigh_ref[...].reshape(BLK_A, K, D_FEAT)
    means = jnp.sum(x, axis=1) * (1.0 / K)
    a = jnp.dot(src_ref[...], ws_ref[...], preferred_element_type=jnp.float32)
    b = jnp.dot(means, wn_ref[...], preferred_element_type=jnp.float32)
    out_ref[:, :AGG] = jnp.maximum(a, 0.0)
    out_ref[:, AGG:] = jnp.maximum(b, 0.0)


def _tc_fused(neigh, src, W_src, W_neighbor):
    return pl.pallas_call(
        _fused_body,
        grid=(-(-N_TC // BLK_A),),
        in_specs=[
            pl.BlockSpec((BLK_A * K, D_FEAT), lambda i: (i, 0)),
            pl.BlockSpec((BLK_A, D_FEAT), lambda i: (i, 0)),
            pl.BlockSpec((D_FEAT, AGG), lambda i: (0, 0)),
            pl.BlockSpec((D_FEAT, AGG), lambda i: (0, 0)),
        ],
        out_specs=pl.BlockSpec((BLK_A, 2 * AGG), lambda i: (i, 0)),
        out_shape=jax.ShapeDtypeStruct((N_SRC, 2 * AGG), jnp.float32),
    )(neigh, src, W_src, W_neighbor)


def _proj_body(acc_ref, src_ref, mean_ref, ws_ref, wn_ref, out_ref):
    del acc_ref
    a = jnp.dot(src_ref[...], ws_ref[...], preferred_element_type=jnp.float32)
    b = jnp.dot(mean_ref[...], wn_ref[...], preferred_element_type=jnp.float32)
    out_ref[:, :AGG] = jnp.maximum(a, 0.0)
    out_ref[:, AGG:] = jnp.maximum(b, 0.0)


def _tc_proj(acc, src, means, W_src, W_neighbor):
    return pl.pallas_call(
        _proj_body,
        grid=(N_SC // BLK_B,),
        in_specs=[
            pl.BlockSpec(memory_space=pl.ANY),
            pl.BlockSpec((BLK_B, D_FEAT), lambda i: (i + N_TC // BLK_B, 0)),
            pl.BlockSpec((BLK_B, D_FEAT), lambda i: (i, 0)),
            pl.BlockSpec((D_FEAT, AGG), lambda i: (0, 0)),
            pl.BlockSpec((D_FEAT, AGG), lambda i: (0, 0)),
        ],
        out_specs=pl.BlockSpec((BLK_B, 2 * AGG), lambda i: (i + N_TC // BLK_B, 0)),
        out_shape=jax.ShapeDtypeStruct((N_SRC, 2 * AGG), jnp.float32),
        input_output_aliases={0: 0},
    )(acc, src, means, W_src, W_neighbor)


def kernel(src_vectors, neighbor_vectors, W_src, W_neighbor):
    sc_means = _sc_mean(neighbor_vectors)
    out_tc = _tc_fused(neighbor_vectors, src_vectors, W_src, W_neighbor)
    return _tc_proj(out_tc, src_vectors, sc_means, W_src, W_neighbor)


# speedup vs baseline: 1.0389x; 1.0389x over previous
import functools

import jax
import jax.numpy as jnp
from jax import lax
from jax.experimental import pallas as pl
from jax.experimental.pallas import tpu as pltpu
from jax.experimental.pallas import tpu_sc as plsc

N_SRC = 10000
N_NEIGH = 160000
D_FEAT = 256
AGG = 128
K = N_NEIGH // N_SRC

N_TC = 8800
N_SC = N_SRC - N_TC

NC = 2
NS = 16
NW = NC * NS
L = 16

CH = 8
N_CHUNKS = N_SC // CH
CHUNKS_PER_W = -(-N_CHUNKS // NW)
NBUF = 3

_sc_mesh = plsc.VectorSubcoreMesh(core_axis_name="c", subcore_axis_name="s")


@functools.partial(
    pl.kernel,
    mesh=_sc_mesh,
    out_type=jax.ShapeDtypeStruct((N_SC, D_FEAT), jnp.float32),
    scratch_types=[
        pltpu.VMEM((NBUF, CH * K, D_FEAT), jnp.float32),
        pltpu.VMEM((NBUF, CH, D_FEAT), jnp.float32),
        pltpu.SemaphoreType.DMA,
        pltpu.SemaphoreType.DMA,
        pltpu.SemaphoreType.DMA,
        pltpu.SemaphoreType.DMA,
        pltpu.SemaphoreType.DMA,
        pltpu.SemaphoreType.DMA,
    ],
)
def _sc_mean(neigh_hbm, out_hbm, buf, obuf, si0, si1, si2, so0, so1, so2):
    wid = lax.axis_index("s") * NC + lax.axis_index("c")
    base = wid * CHUNKS_PER_W
    sin = (si0, si1, si2)
    sout = (so0, so1, so2)

    def src_slab(k):
        g = jnp.minimum(base + k, N_CHUNKS - 1)
        return neigh_hbm.at[pl.ds(N_TC * K + g * (CH * K), CH * K)]

    def dst_slab(k):
        g = jnp.minimum(base + k, N_CHUNKS - 1)
        return out_hbm.at[pl.ds(g * CH, CH)]

    def compute_chunk(p):
        bp = buf.at[p]
        op = obuf.at[p]

        def row_body(i, cc):
            r = i * K
            for c in range(D_FEAT // L):
                off = c * L
                acc = bp[r, pl.ds(off, L)]
                for j in range(1, K):
                    acc = acc + bp[r + j, pl.ds(off, L)]
                op[i, pl.ds(off, L)] = acc * (1.0 / K)
            return cc

        lax.fori_loop(0, CH, row_body, 0)

    for p in range(NBUF):
        pltpu.async_copy(src_slab(p), buf.at[p], sin[p])

    T = CHUNKS_PER_W // NBUF
    R = CHUNKS_PER_W % NBUF

    def ring_body(t, carry):
        for p in range(NBUF):
            k = NBUF * t + p
            pltpu.make_async_copy(src_slab(k), buf.at[p], sin[p]).wait()
            @pl.when(t > 0)
            def _():
                pltpu.make_async_copy(obuf.at[p], dst_slab(k), sout[p]).wait()

            compute_chunk(p)
            pltpu.async_copy(obuf.at[p], dst_slab(k), sout[p])
            pltpu.async_copy(src_slab(k + NBUF), buf.at[p], sin[p])
        return carry

    lax.fori_loop(0, T, ring_body, 0)

    for p in range(R):
        k = NBUF * T + p
        pltpu.make_async_copy(src_slab(k), buf.at[p], sin[p]).wait()
        if T > 0:
            pltpu.make_async_copy(obuf.at[p], dst_slab(k), sout[p]).wait()
        compute_chunk(p)
        pltpu.async_copy(obuf.at[p], dst_slab(k), sout[p])

    for p in range(R, NBUF):
        pltpu.make_async_copy(src_slab(p), buf.at[p], sin[p]).wait()
    for p in range(NBUF):
        pltpu.make_async_copy(obuf.at[p], dst_slab(p), sout[p]).wait()


BLK_A = 1000
BLK_B = 400


def _fused_body(neigh_ref, src_ref, ws_ref, wn_ref, out_ref):
    x = neigh_ref[...].reshape(BLK_A, K, D_FEAT)
    means = jnp.sum(x, axis=1) * (1.0 / K)
    a = jnp.dot(src_ref[...], ws_ref[...], preferred_element_type=jnp.float32)
    b = jnp.dot(means, wn_ref[...], preferred_element_type=jnp.float32)
    out_ref[:, :AGG] = jnp.maximum(a, 0.0)
    out_ref[:, AGG:] = jnp.maximum(b, 0.0)


def _tc_fused(neigh, src, W_src, W_neighbor):
    return pl.pallas_call(
        _fused_body,
        grid=(-(-N_TC // BLK_A),),
        in_specs=[
            pl.BlockSpec((BLK_A * K, D_FEAT), lambda i: (i, 0)),
            pl.BlockSpec((BLK_A, D_FEAT), lambda i: (i, 0)),
            pl.BlockSpec((D_FEAT, AGG), lambda i: (0, 0)),
            pl.BlockSpec((D_FEAT, AGG), lambda i: (0, 0)),
        ],
        out_specs=pl.BlockSpec((BLK_A, 2 * AGG), lambda i: (i, 0)),
        out_shape=jax.ShapeDtypeStruct((N_SRC, 2 * AGG), jnp.float32),
    )(neigh, src, W_src, W_neighbor)


def _proj_body(acc_ref, src_ref, mean_ref, ws_ref, wn_ref, out_ref):
    del acc_ref
    a = jnp.dot(src_ref[...], ws_ref[...], preferred_element_type=jnp.float32)
    b = jnp.dot(mean_ref[...], wn_ref[...], preferred_element_type=jnp.float32)
    out_ref[:, :AGG] = jnp.maximum(a, 0.0)
    out_ref[:, AGG:] = jnp.maximum(b, 0.0)


def _tc_proj(acc, src, means, W_src, W_neighbor):
    return pl.pallas_call(
        _proj_body,
        grid=(N_SC // BLK_B,),
        in_specs=[
            pl.BlockSpec(memory_space=pl.ANY),
            pl.BlockSpec((BLK_B, D_FEAT), lambda i: (i + N_TC // BLK_B, 0)),
            pl.BlockSpec((BLK_B, D_FEAT), lambda i: (i, 0)),
            pl.BlockSpec((D_FEAT, AGG), lambda i: (0, 0)),
            pl.BlockSpec((D_FEAT, AGG), lambda i: (0, 0)),
        ],
        out_specs=pl.BlockSpec((BLK_B, 2 * AGG), lambda i: (i + N_TC // BLK_B, 0)),
        out_shape=jax.ShapeDtypeStruct((N_SRC, 2 * AGG), jnp.float32),
        input_output_aliases={0: 0},
    )(acc, src, means, W_src, W_neighbor)


def kernel(src_vectors, neighbor_vectors, W_src, W_neighbor):
    sc_means = _sc_mean(neighbor_vectors)
    out_tc = _tc_fused(neighbor_vectors, src_vectors, W_src, W_neighbor)
    return _tc_proj(out_tc, src_vectors, sc_means, W_src, W_neighbor)


# speedup vs baseline: 1.0547x; 1.0152x over previous
import functools

import jax
import jax.numpy as jnp
from jax import lax
from jax.experimental import pallas as pl
from jax.experimental.pallas import tpu as pltpu
from jax.experimental.pallas import tpu_sc as plsc

N_SRC = 10000
N_NEIGH = 160000
D_FEAT = 256
AGG = 128
K = N_NEIGH // N_SRC

N_TC = 8800
N_SC = N_SRC - N_TC

NC = 2
NS = 16
NW = NC * NS
L = 16

CH = 8
N_CHUNKS = N_SC // CH
CHUNKS_PER_W = -(-N_CHUNKS // NW)
NBUF = 3

_sc_mesh = plsc.VectorSubcoreMesh(core_axis_name="c", subcore_axis_name="s")


@functools.partial(
    pl.kernel,
    mesh=_sc_mesh,
    out_type=jax.ShapeDtypeStruct((N_SC, D_FEAT), jnp.float32),
    scratch_types=[
        pltpu.VMEM((NBUF, CH * K, D_FEAT), jnp.float32),
        pltpu.VMEM((NBUF, CH, D_FEAT), jnp.float32),
        pltpu.SemaphoreType.DMA,
        pltpu.SemaphoreType.DMA,
        pltpu.SemaphoreType.DMA,
        pltpu.SemaphoreType.DMA,
        pltpu.SemaphoreType.DMA,
        pltpu.SemaphoreType.DMA,
    ],
)
def _sc_mean(neigh_hbm, out_hbm, buf, obuf, si0, si1, si2, so0, so1, so2):
    wid = lax.axis_index("s") * NC + lax.axis_index("c")
    base = wid * CHUNKS_PER_W
    sin = (si0, si1, si2)
    sout = (so0, so1, so2)

    def src_slab(k):
        g = jnp.minimum(base + k, N_CHUNKS - 1)
        return neigh_hbm.at[pl.ds(N_TC * K + g * (CH * K), CH * K)]

    def dst_slab(k):
        g = jnp.minimum(base + k, N_CHUNKS - 1)
        return out_hbm.at[pl.ds(g * CH, CH)]

    def compute_chunk(p):
        bp = buf.at[p]
        op = obuf.at[p]

        def row_body(i, cc):
            r = i * K
            for c in range(D_FEAT // L):
                off = c * L
                acc = bp[r, pl.ds(off, L)]
                for j in range(1, K):
                    acc = acc + bp[r + j, pl.ds(off, L)]
                op[i, pl.ds(off, L)] = acc * (1.0 / K)
            return cc

        lax.fori_loop(0, CH, row_body, 0)

    for p in range(NBUF):
        pltpu.async_copy(src_slab(p), buf.at[p], sin[p])

    T = CHUNKS_PER_W // NBUF
    R = CHUNKS_PER_W % NBUF

    def ring_body(t, carry):
        for p in range(NBUF):
            k = NBUF * t + p
            pltpu.make_async_copy(src_slab(k), buf.at[p], sin[p]).wait()
            @pl.when(t > 0)
            def _():
                pltpu.make_async_copy(obuf.at[p], dst_slab(k), sout[p]).wait()

            compute_chunk(p)
            pltpu.async_copy(obuf.at[p], dst_slab(k), sout[p])
            pltpu.async_copy(src_slab(k + NBUF), buf.at[p], sin[p])
        return carry

    lax.fori_loop(0, T, ring_body, 0)

    for p in range(R):
        k = NBUF * T + p
        pltpu.make_async_copy(src_slab(k), buf.at[p], sin[p]).wait()
        if T > 0:
            pltpu.make_async_copy(obuf.at[p], dst_slab(k), sout[p]).wait()
        compute_chunk(p)
        pltpu.async_copy(obuf.at[p], dst_slab(k), sout[p])

    for p in range(R, NBUF):
        pltpu.make_async_copy(src_slab(p), buf.at[p], sin[p]).wait()
    for p in range(NBUF):
        pltpu.make_async_copy(obuf.at[p], dst_slab(p), sout[p]).wait()


BLK_A = 880
BLK_B = 400


def _fused_body(neigh_ref, src_ref, ws_ref, wn_ref, out_ref):
    x = neigh_ref[...].reshape(BLK_A, K, D_FEAT)
    means = jnp.sum(x, axis=1) * (1.0 / K)
    a = jnp.dot(src_ref[...], ws_ref[...], preferred_element_type=jnp.float32)
    b = jnp.dot(means, wn_ref[...], preferred_element_type=jnp.float32)
    out_ref[:, :AGG] = jnp.maximum(a, 0.0)
    out_ref[:, AGG:] = jnp.maximum(b, 0.0)


def _tc_fused(neigh, src, W_src, W_neighbor):
    return pl.pallas_call(
        _fused_body,
        grid=(-(-N_TC // BLK_A),),
        in_specs=[
            pl.BlockSpec((BLK_A * K, D_FEAT), lambda i: (i, 0)),
            pl.BlockSpec((BLK_A, D_FEAT), lambda i: (i, 0)),
            pl.BlockSpec((D_FEAT, AGG), lambda i: (0, 0)),
            pl.BlockSpec((D_FEAT, AGG), lambda i: (0, 0)),
        ],
        out_specs=pl.BlockSpec((BLK_A, 2 * AGG), lambda i: (i, 0)),
        out_shape=jax.ShapeDtypeStruct((N_SRC, 2 * AGG), jnp.float32),
    )(neigh, src, W_src, W_neighbor)


def _proj_body(acc_ref, src_ref, mean_ref, ws_ref, wn_ref, out_ref):
    del acc_ref
    a = jnp.dot(src_ref[...], ws_ref[...], preferred_element_type=jnp.float32)
    b = jnp.dot(mean_ref[...], wn_ref[...], preferred_element_type=jnp.float32)
    out_ref[:, :AGG] = jnp.maximum(a, 0.0)
    out_ref[:, AGG:] = jnp.maximum(b, 0.0)


def _tc_proj(acc, src, means, W_src, W_neighbor):
    return pl.pallas_call(
        _proj_body,
        grid=(N_SC // BLK_B,),
        in_specs=[
            pl.BlockSpec(memory_space=pl.ANY),
            pl.BlockSpec((BLK_B, D_FEAT), lambda i: (i + N_TC // BLK_B, 0)),
            pl.BlockSpec((BLK_B, D_FEAT), lambda i: (i, 0)),
            pl.BlockSpec((D_FEAT, AGG), lambda i: (0, 0)),
            pl.BlockSpec((D_FEAT, AGG), lambda i: (0, 0)),
        ],
        out_specs=pl.BlockSpec((BLK_B, 2 * AGG), lambda i: (i + N_TC // BLK_B, 0)),
        out_shape=jax.ShapeDtypeStruct((N_SRC, 2 * AGG), jnp.float32),
        input_output_aliases={0: 0},
    )(acc, src, means, W_src, W_neighbor)


def kernel(src_vectors, neighbor_vectors, W_src, W_neighbor):
    sc_means = _sc_mean(neighbor_vectors)
    out_tc = _tc_fused(neighbor_vectors, src_vectors, W_src, W_neighbor)
    return _tc_proj(out_tc, src_vectors, sc_means, W_src, W_neighbor)
